# baseline (speedup 1.0000x reference)
import functools

import jax
import jax.numpy as jnp
from jax import lax
from jax.experimental import pallas as pl
from jax.experimental.pallas import tpu as pltpu
from jax.experimental.pallas import tpu_sc as plsc

NUM_ROWS = 1000000
DIM = 64
ROWS_PER_BLOCK = 8000
NUM_BLOCKS = NUM_ROWS // ROWS_PER_BLOCK

NW = 32
XROWS = 16384
XCOLS = 26
XR_PER_W = XROWS // NW
G_XR = 4
CHUNK = G_XR * XCOLS
NCHUNK = XR_PER_W // G_XR
PER_W = XR_PER_W * XCOLS


def _sigma_body(w_ref, u_ref, pack_ref, o_ref, g_acc, t_acc):
    i = pl.program_id(0)

    @pl.when(i == 0)
    def _init():
        g_acc[...] = jnp.zeros_like(g_acc)
        t_acc[...] = jnp.zeros_like(t_acc)

    w = w_ref[...]
    half = ROWS_PER_BLOCK // 2
    pack_ref[...] = jnp.concatenate([w[:half, :], w[half:, :]], axis=1)
    u = u_ref[...].reshape(1, ROWS_PER_BLOCK)
    g_acc[...] += lax.dot_general(
        w, w, (((0,), (0,)), ((), ())), preferred_element_type=jnp.float32)
    t_acc[...] += jnp.dot(u, w, preferred_element_type=jnp.float32)

    @pl.when(i == NUM_BLOCKS - 1)
    def _fini():
        eps = 1e-12
        t = t_acc[...]
        nt = jnp.sqrt(jnp.sum(t * t))
        v = t / (nt + eps)
        gv = jnp.dot(v, g_acc[...], preferred_element_type=jnp.float32)
        s2 = jnp.sum(gv * v)
        sigma = s2 / (jnp.sqrt(s2) + eps)
        o_ref[...] = jnp.broadcast_to(1.0 / sigma, (8, 128))


def _sigma_pack(weight, u):
    return pl.pallas_call(
        _sigma_body,
        grid=(NUM_BLOCKS,),
        in_specs=[
            pl.BlockSpec((ROWS_PER_BLOCK, DIM), lambda i: (i, 0)),
            pl.BlockSpec((1, 1, ROWS_PER_BLOCK), lambda i: (i, 0, 0)),
        ],
        out_specs=[
            pl.BlockSpec((ROWS_PER_BLOCK // 2, 2 * DIM), lambda i: (i, 0)),
            pl.BlockSpec((8, 128), lambda i: (0, 0)),
        ],
        out_shape=[
            jax.ShapeDtypeStruct((NUM_ROWS // 2, 2 * DIM), jnp.float32),
            jax.ShapeDtypeStruct((8, 128), jnp.float32),
        ],
        scratch_shapes=[
            pltpu.VMEM((DIM, DIM), jnp.float32),
            pltpu.VMEM((1, DIM), jnp.float32),
        ],
    )(weight, u.reshape(NUM_BLOCKS, 1, ROWS_PER_BLOCK))


@functools.cache
def _make_gather():
    mesh = plsc.VectorSubcoreMesh(
        core_axis_name="c", subcore_axis_name="s", num_cores=2, num_subcores=16)

    @functools.partial(
        pl.kernel,
        out_type=jax.ShapeDtypeStruct((XROWS * XCOLS, DIM), jnp.float32),
        mesh=mesh,
        scratch_types=[
            pltpu.VMEM((PER_W + 16,), jnp.int32),
            pltpu.VMEM((PER_W + 16,), jnp.int32),
            pltpu.VMEM((16,), jnp.float32),
            pltpu.VMEM((CHUNK, 2 * DIM), jnp.float32),
            pltpu.VMEM((CHUNK, DIM), jnp.float32),
            pltpu.SemaphoreType.DMA,
        ],
        compiler_params=pltpu.CompilerParams(needs_layout_passes=False),
    )
    def _gather(pack_hbm, pidx_hbm, colb_hbm, scale_hbm, out_hbm,
                pidx_v, colb_v, scale_v, gbuf, outb, sem):
        wid = lax.axis_index("s") * 2 + lax.axis_index("c")
        base = wid * PER_W
        pltpu.sync_copy(pidx_hbm.at[pl.ds(base, PER_W)],
                        pidx_v.at[pl.ds(0, PER_W)])
        pltpu.sync_copy(colb_hbm.at[pl.ds(base, PER_W)],
                        colb_v.at[pl.ds(0, PER_W)])
        pltpu.sync_copy(scale_hbm, scale_v)
        s = scale_v[...]
        xrow0 = wid * XR_PER_W

        def chunk(c, carry):
            off = c * CHUNK
            pltpu.async_copy(
                pack_hbm.at[pidx_v.at[pl.ds(off, CHUNK)]], gbuf, sem).wait()
            for g in range(7):
                pos = lax.iota(jnp.int32, 16) + (16 * g)
                m = pos < CHUNK if g == 6 else None
                cb = colb_v[pl.ds(off + 16 * g, 16)]

                def col(d, cc):
                    dv = jnp.full((16,), d, jnp.int32)
                    val = plsc.load_gather(gbuf, [pos, cb + d], mask=m)
                    plsc.store_scatter(outb, [pos, dv], val * s, mask=m)
                    return cc

                lax.fori_loop(0, DIM, col, 0, unroll=8)
            pltpu.sync_copy(outb, out_hbm.at[pl.ds(base + off, CHUNK)])
            return carry

        lax.fori_loop(0, NCHUNK, chunk, 0)

    return _gather


def kernel(x, weight, u):
    wpack, inv_blk = _sigma_pack(weight, u)
    scale16 = inv_blk[0, :16]
    half = ROWS_PER_BLOCK // 2
    pidx = ((x // ROWS_PER_BLOCK) * half + (x % half)).reshape(-1)
    colb = jnp.where((x % ROWS_PER_BLOCK) >= half, DIM, 0).astype(
        jnp.int32).reshape(-1)
    out2 = _make_gather()(wpack, pidx, colb, scale16)
    return out2.reshape(XROWS, XCOLS, DIM)


# speedup vs baseline: 1.7167x; 1.7167x over previous
import functools

import jax
import jax.numpy as jnp
from jax import lax
from jax.experimental import pallas as pl
from jax.experimental.pallas import tpu as pltpu
from jax.experimental.pallas import tpu_sc as plsc

NUM_ROWS = 1000000
DIM = 64
ROWS_PER_BLOCK = 8000
NUM_BLOCKS = NUM_ROWS // ROWS_PER_BLOCK

NW = 32
XROWS = 16384
XCOLS = 26
XR_PER_W = XROWS // NW
G_XR = 4
CHUNK = G_XR * XCOLS
NCHUNK = XR_PER_W // G_XR
PER_W = XR_PER_W * XCOLS


def _sigma_body(w_ref, u_ref, pack_ref, o_ref, g_acc, t_acc):
    i = pl.program_id(0)

    @pl.when(i == 0)
    def _init():
        g_acc[...] = jnp.zeros_like(g_acc)
        t_acc[...] = jnp.zeros_like(t_acc)

    w = w_ref[...]
    half = ROWS_PER_BLOCK // 2
    pack_ref[...] = jnp.concatenate([w[:half, :], w[half:, :]], axis=1)
    u = u_ref[...].reshape(1, ROWS_PER_BLOCK)
    g_acc[...] += lax.dot_general(
        w, w, (((0,), (0,)), ((), ())), preferred_element_type=jnp.float32)
    t_acc[...] += jnp.dot(u, w, preferred_element_type=jnp.float32)

    @pl.when(i == NUM_BLOCKS - 1)
    def _fini():
        eps = 1e-12
        t = t_acc[...]
        nt = jnp.sqrt(jnp.sum(t * t))
        v = t / (nt + eps)
        gv = jnp.dot(v, g_acc[...], preferred_element_type=jnp.float32)
        s2 = jnp.sum(gv * v)
        sigma = s2 / (jnp.sqrt(s2) + eps)
        o_ref[...] = jnp.broadcast_to(1.0 / sigma, (8, 128))


def _sigma_pack(weight, u):
    return pl.pallas_call(
        _sigma_body,
        grid=(NUM_BLOCKS,),
        in_specs=[
            pl.BlockSpec((ROWS_PER_BLOCK, DIM), lambda i: (i, 0)),
            pl.BlockSpec((1, 1, ROWS_PER_BLOCK), lambda i: (i, 0, 0)),
        ],
        out_specs=[
            pl.BlockSpec((ROWS_PER_BLOCK // 2, 2 * DIM), lambda i: (i, 0)),
            pl.BlockSpec((8, 128), lambda i: (0, 0)),
        ],
        out_shape=[
            jax.ShapeDtypeStruct((NUM_ROWS // 2, 2 * DIM), jnp.float32),
            jax.ShapeDtypeStruct((8, 128), jnp.float32),
        ],
        scratch_shapes=[
            pltpu.VMEM((DIM, DIM), jnp.float32),
            pltpu.VMEM((1, DIM), jnp.float32),
        ],
    )(weight, u.reshape(NUM_BLOCKS, 1, ROWS_PER_BLOCK))


@functools.cache
def _make_gather():
    mesh = plsc.VectorSubcoreMesh(
        core_axis_name="c", subcore_axis_name="s", num_cores=2, num_subcores=16)

    NBUF = 4

    @functools.partial(
        pl.kernel,
        out_type=jax.ShapeDtypeStruct((XROWS * XCOLS, DIM), jnp.float32),
        mesh=mesh,
        scratch_types=[
            pltpu.VMEM((PER_W,), jnp.int32),
            pltpu.VMEM((PER_W,), jnp.int32),
            pltpu.VMEM((16,), jnp.float32),
            [pltpu.VMEM((CHUNK, 2 * DIM), jnp.float32) for _ in range(NBUF)],
            [pltpu.VMEM((CHUNK, DIM), jnp.float32) for _ in range(2)],
            [pltpu.SemaphoreType.DMA for _ in range(NBUF)],
            [pltpu.SemaphoreType.DMA for _ in range(2)],
        ],
        compiler_params=pltpu.CompilerParams(needs_layout_passes=False),
    )
    def _gather(pack_hbm, pidx_hbm, colb_hbm, scale_hbm, out_hbm,
                pidx_v, colb_v, scale_v, gbufs, outbs, gsems, osems):
        wid = lax.axis_index("s") * 2 + lax.axis_index("c")
        base = wid * PER_W
        pltpu.sync_copy(pidx_hbm.at[pl.ds(base, PER_W)], pidx_v)
        pltpu.sync_copy(colb_hbm.at[pl.ds(base, PER_W)], colb_v)
        pltpu.sync_copy(scale_hbm, scale_v)
        s = scale_v[...]

        def gather_chunk(c, b):
            pltpu.async_copy(
                pack_hbm.at[pidx_v.at[pl.ds(c * CHUNK, CHUNK)]],
                gbufs[b], gsems[b])

        def wait_gather(b):
            pltpu.make_async_copy(
                pack_hbm.at[pidx_v.at[pl.ds(0, CHUNK)]],
                gbufs[b], gsems[b]).wait()

        def put_chunk(c, b):
            pltpu.async_copy(outbs[b % 2],
                             out_hbm.at[pl.ds(base + c * CHUNK, CHUNK)],
                             osems[b % 2])

        def wait_put(b):
            pltpu.make_async_copy(outbs[b % 2],
                                  out_hbm.at[pl.ds(base, CHUNK)],
                                  osems[b % 2]).wait()

        def select_chunk(c, b):
            gb, ob = gbufs[b], outbs[b % 2]
            off = c * CHUNK

            def row(r, carry):
                pb = plsc.load_gather(colb_v, [jnp.full((16,), off, jnp.int32) + r])
                m = pb > 0
                for k in range(DIM // 16):
                    lo = gb[r, pl.ds(16 * k, 16)]
                    hi = gb[r, pl.ds(DIM + 16 * k, 16)]
                    ob[r, pl.ds(16 * k, 16)] = jnp.where(m, hi, lo) * s
                return carry

            lax.fori_loop(0, CHUNK, row, 0, unroll=2)

        for b in range(NBUF):
            gather_chunk(b, b)

        def step(i, carry):
            for b in range(NBUF):
                c = i * NBUF + b
                wait_gather(b)

                @pl.when((i > 0) | (b >= 2))
                def _drain():
                    wait_put(b)

                select_chunk(c, b)
                put_chunk(c, b)

                @pl.when(c + NBUF < NCHUNK)
                def _prefetch():
                    gather_chunk(c + NBUF, b)
            return carry

        lax.fori_loop(0, NCHUNK // NBUF, step, 0)
        for b in range(2):
            wait_put(b)

    return _gather


def kernel(x, weight, u):
    wpack, inv_blk = _sigma_pack(weight, u)
    scale16 = inv_blk[0, :16]
    half = ROWS_PER_BLOCK // 2
    pidx = ((x // ROWS_PER_BLOCK) * half + (x % half)).reshape(-1)
    colb = jnp.where((x % ROWS_PER_BLOCK) >= half, DIM, 0).astype(
        jnp.int32).reshape(-1)
    out2 = _make_gather()(wpack, pidx, colb, scale16)
    return out2.reshape(XROWS, XCOLS, DIM)


# speedup vs baseline: 1.8893x; 1.1005x over previous
import functools

import jax
import jax.numpy as jnp
from jax import lax
from jax.experimental import pallas as pl
from jax.experimental.pallas import tpu as pltpu
from jax.experimental.pallas import tpu_sc as plsc

NUM_ROWS = 1000000
DIM = 64
ROWS_PER_BLOCK = 8000
NUM_BLOCKS = NUM_ROWS // ROWS_PER_BLOCK

NW = 32
XROWS = 16384
XCOLS = 26
XR_PER_W = XROWS // NW
G_XR = 4
CHUNK = G_XR * XCOLS
NCHUNK = XR_PER_W // G_XR
PER_W = XR_PER_W * XCOLS


def _sigma_body(w_ref, u_ref, pack_ref, o_ref, g_acc, t_acc):
    i = pl.program_id(0)

    @pl.when(i == 0)
    def _init():
        g_acc[...] = jnp.zeros_like(g_acc)
        t_acc[...] = jnp.zeros_like(t_acc)

    w = w_ref[...]
    half = ROWS_PER_BLOCK // 2
    pack_ref[...] = jnp.concatenate([w[:half, :], w[half:, :]], axis=1)
    u = u_ref[...].reshape(1, ROWS_PER_BLOCK)
    g_acc[...] += lax.dot_general(
        w, w, (((0,), (0,)), ((), ())), preferred_element_type=jnp.float32)
    t_acc[...] += jnp.dot(u, w, preferred_element_type=jnp.float32)

    @pl.when(i == NUM_BLOCKS - 1)
    def _fini():
        eps = 1e-12
        t = t_acc[...]
        nt = jnp.sqrt(jnp.sum(t * t))
        v = t / (nt + eps)
        gv = jnp.dot(v, g_acc[...], preferred_element_type=jnp.float32)
        s2 = jnp.sum(gv * v)
        sigma = s2 / (jnp.sqrt(s2) + eps)
        o_ref[...] = jnp.broadcast_to(1.0 / sigma, (8, 128))


def _sigma_pack(weight, u):
    return pl.pallas_call(
        _sigma_body,
        grid=(NUM_BLOCKS,),
        in_specs=[
            pl.BlockSpec((ROWS_PER_BLOCK, DIM), lambda i: (i, 0)),
            pl.BlockSpec((1, 1, ROWS_PER_BLOCK), lambda i: (i, 0, 0)),
        ],
        out_specs=[
            pl.BlockSpec((ROWS_PER_BLOCK // 2, 2 * DIM), lambda i: (i, 0)),
            pl.BlockSpec((8, 128), lambda i: (0, 0)),
        ],
        out_shape=[
            jax.ShapeDtypeStruct((NUM_ROWS // 2, 2 * DIM), jnp.float32),
            jax.ShapeDtypeStruct((8, 128), jnp.float32),
        ],
        scratch_shapes=[
            pltpu.VMEM((DIM, DIM), jnp.float32),
            pltpu.VMEM((1, DIM), jnp.float32),
        ],
    )(weight, u.reshape(NUM_BLOCKS, 1, ROWS_PER_BLOCK))


@functools.cache
def _make_gather():
    mesh = plsc.VectorSubcoreMesh(
        core_axis_name="c", subcore_axis_name="s", num_cores=2, num_subcores=16)

    NBUF = 4

    @functools.partial(
        pl.kernel,
        out_type=jax.ShapeDtypeStruct((XROWS, XCOLS, DIM), jnp.float32),
        mesh=mesh,
        scratch_types=[
            pltpu.VMEM((PER_W,), jnp.int32),
            pltpu.VMEM((PER_W,), jnp.int32),
            pltpu.VMEM((16,), jnp.float32),
            [pltpu.VMEM((CHUNK, 2 * DIM), jnp.float32) for _ in range(NBUF)],
            [pltpu.VMEM((CHUNK, DIM), jnp.float32) for _ in range(2)],
            [pltpu.SemaphoreType.DMA for _ in range(NBUF)],
            [pltpu.SemaphoreType.DMA for _ in range(2)],
        ],
        compiler_params=pltpu.CompilerParams(needs_layout_passes=False),
    )
    def _gather(pack_hbm, pidx_hbm, colb_hbm, scale_hbm, out_hbm,
                pidx_v, colb_v, scale_v, gbufs, outbs, gsems, osems):
        wid = lax.axis_index("s") * 2 + lax.axis_index("c")
        base = wid * PER_W
        pltpu.sync_copy(pidx_hbm.at[pl.ds(base, PER_W)], pidx_v)
        pltpu.sync_copy(colb_hbm.at[pl.ds(base, PER_W)], colb_v)
        pltpu.sync_copy(scale_hbm, scale_v)
        s = scale_v[...]

        def gather_chunk(c, b):
            pltpu.async_copy(
                pack_hbm.at[pidx_v.at[pl.ds(c * CHUNK, CHUNK)]],
                gbufs[b], gsems[b])

        def wait_gather(b):
            pltpu.make_async_copy(
                pack_hbm.at[pidx_v.at[pl.ds(0, CHUNK)]],
                gbufs[b], gsems[b]).wait()

        xrow0 = wid * XR_PER_W

        def put_chunk(c, b):
            for j in range(G_XR):
                pltpu.async_copy(outbs[b % 2].at[pl.ds(j * XCOLS, XCOLS)],
                                 out_hbm.at[xrow0 + c * G_XR + j],
                                 osems[b % 2])

        def wait_put(b):
            for j in range(G_XR):
                pltpu.make_async_copy(outbs[b % 2].at[pl.ds(0, XCOLS)],
                                      out_hbm.at[xrow0],
                                      osems[b % 2]).wait()

        def select_chunk(c, b):
            gb, ob = gbufs[b], outbs[b % 2]
            off = c * CHUNK

            def row(r, carry):
                pb = plsc.load_gather(colb_v, [jnp.full((16,), off, jnp.int32) + r])
                m = pb > 0
                for k in range(DIM // 16):
                    lo = gb[r, pl.ds(16 * k, 16)]
                    hi = gb[r, pl.ds(DIM + 16 * k, 16)]
                    ob[r, pl.ds(16 * k, 16)] = jnp.where(m, hi, lo) * s
                return carry

            lax.fori_loop(0, CHUNK, row, 0, unroll=2)

        for b in range(NBUF):
            gather_chunk(b, b)

        def step(i, carry):
            for b in range(NBUF):
                c = i * NBUF + b
                wait_gather(b)

                @pl.when((i > 0) | (b >= 2))
                def _drain():
                    wait_put(b)

                select_chunk(c, b)
                put_chunk(c, b)

                @pl.when(c + NBUF < NCHUNK)
                def _prefetch():
                    gather_chunk(c + NBUF, b)
            return carry

        lax.fori_loop(0, NCHUNK // NBUF, step, 0)
        for b in range(2):
            wait_put(b)

    return _gather


def kernel(x, weight, u):
    wpack, inv_blk = _sigma_pack(weight, u)
    scale16 = inv_blk[0, :16]
    half = ROWS_PER_BLOCK // 2
    pidx = ((x // ROWS_PER_BLOCK) * half + (x % half)).reshape(-1)
    colb = jnp.where((x % ROWS_PER_BLOCK) >= half, DIM, 0).astype(
        jnp.int32).reshape(-1)
    return _make_gather()(wpack, pidx, colb, scale16)
